# Initial kernel scaffold; baseline (speedup 1.0000x reference)
import jax
import jax.numpy as jnp
from jax.experimental import pallas as pl


def kernel(input_tensor, W, b):
    raise NotImplementedError("write your pallas kernel here")


# speedup vs baseline: 28.1837x; 28.1837x over previous
import jax
import jax.numpy as jnp
from jax.experimental import pallas as pl
from jax.experimental.pallas import tpu as pltpu

B, T, CIN, N = 4, 16, 68, 128
HD = 128
K = 16
NG = 4 * HD
GR = NG + HD

_MM = (((1,), (0,)), ((), ()))
_CC = (((0,), (0,)), ((), ()))


def _step_kernel(xc_ref, xp_ref, xpt_ref, wx_ref, woh_ref, bias_ref, out_ref,
                 h_ref, c_ref):
    t = pl.program_id(0)
    bi = pl.program_id(1)
    f32 = jnp.float32

    xt = xc_ref[0, 0]
    pos_t = xt[:4, :]
    pos_prev = xp_ref[0, 0]

    first = t == 0
    h_prev = jnp.where(first, 0.0, h_ref[bi])
    c_prev = jnp.where(first, 0.0, c_ref[bi])

    A = jax.lax.dot_general(wx_ref[...], xt, _MM,
                            preferred_element_type=f32) + bias_ref[...]
    ph = jnp.concatenate([pos_prev, h_prev], axis=0)
    Bm = jax.lax.dot_general(woh_ref[...], ph, _MM,
                             preferred_element_type=f32)
    G = jnp.concatenate([Bm, c_prev], axis=0)

    ppt = xpt_ref[0, 0]
    E = jnp.zeros((N, N), f32)
    for ci in range(4):
        d = ppt[:, ci:ci + 1] - pos_t[ci:ci + 1, :]
        E = E + d * d

    iota_m = jax.lax.broadcasted_iota(jnp.int32, (N, N), 0)
    BIG = f32(3.0e38)
    hmax = jnp.full((HD, N), -BIG, f32)
    cmax = jnp.full((HD, N), -BIG, f32)
    for _ in range(K):
        v = jnp.min(E, axis=0, keepdims=True)
        eq = E == v
        am = jnp.min(jnp.where(eq, iota_m, N), axis=0,
                     keepdims=True)
        sel = iota_m == am
        E = jnp.where(sel, BIG, E)
        onehot = sel.astype(f32)
        Gk = jax.lax.dot_general(G, onehot, _MM,
                                 preferred_element_type=f32)
        g = A + Gk[:NG]
        i_g = jax.nn.sigmoid(g[0:HD])
        f_g = jax.nn.sigmoid(g[HD:2 * HD])
        o_g = jax.nn.sigmoid(g[2 * HD:3 * HD])
        g_g = jnp.tanh(g[3 * HD:4 * HD])
        cn = f_g * Gk[NG:] + i_g * g_g
        hn = o_g * jnp.tanh(cn)
        hmax = jnp.maximum(hmax, hn)
        cmax = jnp.maximum(cmax, cn)

    h_ref[bi] = hmax
    c_ref[bi] = cmax
    out_ref[0, 0] = hmax


def _run(xs, pos, post, wx, woh, bias2):
    return pl.pallas_call(
        _step_kernel,
        grid=(T, B),
        in_specs=[
            pl.BlockSpec((1, 1, CIN, N), lambda t, b: (t, b, 0, 0)),
            pl.BlockSpec((1, 1, 4, N),
                         lambda t, b: (jnp.maximum(t - 1, 0), b, 0, 0)),
            pl.BlockSpec((1, 1, N, 4),
                         lambda t, b: (jnp.maximum(t - 1, 0), b, 0, 0)),
            pl.BlockSpec((NG, CIN), lambda t, b: (0, 0)),
            pl.BlockSpec((NG, 4 + HD), lambda t, b: (0, 0)),
            pl.BlockSpec((NG, N), lambda t, b: (0, 0)),
        ],
        out_specs=pl.BlockSpec((1, 1, HD, N), lambda t, b: (t, b, 0, 0)),
        out_shape=jax.ShapeDtypeStruct((T, B, HD, N), jnp.float32),
        scratch_shapes=[
            pltpu.VMEM((B, HD, N), jnp.float32),
            pltpu.VMEM((B, HD, N), jnp.float32),
        ],
        compiler_params=pltpu.CompilerParams(
            dimension_semantics=("arbitrary", "arbitrary")),
    )(xs, pos, post, wx, woh, bias2)


def kernel(input_tensor, W, b):
    x = input_tensor
    xs = x.transpose(1, 0, 2, 3)
    pos = xs[:, :, :4, :]
    post = pos.transpose(0, 1, 3, 2)
    wx = W[:, :CIN].at[:, :4].add(-W[:, CIN:CIN + 4])
    woh = W[:, CIN:]
    bias2 = jnp.broadcast_to(b[:, None], (NG, N))
    h_out = _run(xs, pos, post, wx, woh, bias2)
    return jnp.concatenate([x[:, :, :4, :], h_out.transpose(1, 0, 2, 3)],
                           axis=2)


# speedup vs baseline: 33.8733x; 1.2019x over previous
import jax
import jax.numpy as jnp
from jax.experimental import pallas as pl
from jax.experimental.pallas import tpu as pltpu

B, T, CIN, N = 4, 16, 68, 128
HD = 128
K = 16
NG = 4 * HD
GR = NG + HD

_MM = (((1,), (0,)), ((), ()))


def _lstm_kernel(x_ref, post_ref, wx_ref, woh_ref, bias_ref, out_ref,
                 h_ref, c_ref):
    f32 = jnp.float32
    iota_m = jax.lax.broadcasted_iota(jnp.int32, (N, N), 0)
    BIG = f32(3.0e38)

    def step(t, carry):
        tp = jnp.maximum(t - 1, 0)
        first = t == 0
        for b in range(B):
            xt = x_ref[t, b]
            pos_t = xt[:4, :]
            pos_prev = x_ref[tp, b, :4, :]
            ppt = post_ref[tp, b]
            h_prev = jnp.where(first, 0.0, h_ref[b])
            c_prev = jnp.where(first, 0.0, c_ref[b])

            A = jax.lax.dot_general(wx_ref[...], xt, _MM,
                                    preferred_element_type=f32)
            A = A + bias_ref[...]
            ph = jnp.concatenate([pos_prev, h_prev], axis=0)
            Bm = jax.lax.dot_general(woh_ref[...], ph, _MM,
                                     preferred_element_type=f32)
            G = jnp.concatenate([Bm, c_prev], axis=0)

            E = jnp.zeros((N, N), f32)
            for ci in range(4):
                d = ppt[:, ci:ci + 1] - pos_t[ci:ci + 1, :]
                E = E + d * d

            hmax = jnp.full((HD, N), -BIG, f32)
            cmax = jnp.full((HD, N), -BIG, f32)
            for _ in range(K):
                v = jnp.min(E, axis=0, keepdims=True)
                eq = E == v
                am = jnp.min(jnp.where(eq, iota_m, N), axis=0,
                             keepdims=True)
                sel = iota_m == am
                E = jnp.where(sel, BIG, E)
                onehot = sel.astype(f32)
                Gk = jax.lax.dot_general(G, onehot, _MM,
                                         preferred_element_type=f32)
                g = A + Gk[:NG]
                th_i = jnp.tanh(g[0:HD])
                th_f = jnp.tanh(g[HD:2 * HD])
                th_o = jnp.tanh(g[2 * HD:3 * HD])
                t_g = jnp.tanh(g[3 * HD:4 * HD])
                cg = Gk[NG:]
                cn = 0.5 * ((th_f + 1.0) * cg + (th_i + 1.0) * t_g)
                hn = (0.5 * th_o + 0.5) * jnp.tanh(cn)
                hmax = jnp.maximum(hmax, hn)
                cmax = jnp.maximum(cmax, cn)

            h_ref[b] = hmax
            c_ref[b] = cmax
            out_ref[t, b] = hmax
        return carry

    jax.lax.fori_loop(0, T, step, 0)


def _run(xs, post, wx, woh, bias2):
    return pl.pallas_call(
        _lstm_kernel,
        in_specs=[
            pl.BlockSpec((T, B, CIN, N), lambda: (0, 0, 0, 0)),
            pl.BlockSpec((T, B, N, 4), lambda: (0, 0, 0, 0)),
            pl.BlockSpec((NG, CIN), lambda: (0, 0)),
            pl.BlockSpec((NG, 4 + HD), lambda: (0, 0)),
            pl.BlockSpec((NG, N), lambda: (0, 0)),
        ],
        out_specs=pl.BlockSpec((T, B, HD, N), lambda: (0, 0, 0, 0)),
        out_shape=jax.ShapeDtypeStruct((T, B, HD, N), jnp.float32),
        scratch_shapes=[
            pltpu.VMEM((B, HD, N), jnp.float32),
            pltpu.VMEM((B, HD, N), jnp.float32),
        ],
    )(xs, post, wx, woh, bias2)


def kernel(input_tensor, W, b):
    x = input_tensor
    xs = x.transpose(1, 0, 2, 3)
    post = xs[:, :, :4, :].transpose(0, 1, 3, 2)
    wx = W[:, :CIN].at[:, :4].add(-W[:, CIN:CIN + 4])
    woh = W[:, CIN:]
    scale = jnp.concatenate([jnp.full((3 * HD, 1), 0.5, jnp.float32),
                             jnp.ones((HD, 1), jnp.float32)], axis=0)
    wx = wx * scale
    woh = woh * scale
    bias2 = jnp.broadcast_to(b[:, None] * scale, (NG, N))
    h_out = _run(xs, post, wx, woh, bias2)
    return jnp.concatenate([x[:, :, :4, :], h_out.transpose(1, 0, 2, 3)],
                           axis=2)
